# baseline (speedup 1.0000x reference)
import jax
import jax.numpy as jnp
from jax.experimental import pallas as pl
from jax.experimental.pallas import tpu as pltpu

_ROWS, _COLS = 1024, 100000
_BLOCK_ROWS = 8
_GRID = _ROWS // _BLOCK_ROWS


def _cw_kernel(y_ref, x_ref, out_ref):
    x = x_ref[...]
    yb = y_ref[0, 0, :]
    cols = jax.lax.broadcasted_iota(jnp.int32, x.shape, 1)
    is_y = cols == yb[:, None]
    neg = jnp.float32(-jnp.inf)
    best_other = jnp.max(jnp.where(is_y, neg, x), axis=-1)
    correct = jnp.max(jnp.where(is_y, x, neg), axis=-1)
    out_ref[0, 0, :] = jnp.maximum(correct - best_other, 0.0)


@jax.jit
def kernel(logits, y):
    y3 = y.astype(jnp.int32).reshape(_GRID, 1, _BLOCK_ROWS)
    out = pl.pallas_call(
        _cw_kernel,
        grid=(_GRID,),
        in_specs=[
            pl.BlockSpec((1, 1, _BLOCK_ROWS), lambda i: (i, 0, 0)),
            pl.BlockSpec((_BLOCK_ROWS, _COLS), lambda i: (i, 0)),
        ],
        out_specs=pl.BlockSpec((1, 1, _BLOCK_ROWS), lambda i: (i, 0, 0)),
        out_shape=jax.ShapeDtypeStruct((_GRID, 1, _BLOCK_ROWS), logits.dtype),
        compiler_params=pltpu.CompilerParams(
            dimension_semantics=("parallel",)
        ),
    )(y3, logits)
    return out.reshape(_ROWS)


# speedup vs baseline: 1.0884x; 1.0884x over previous
import jax
import jax.numpy as jnp
from jax import lax
from jax.experimental import pallas as pl
from jax.experimental.pallas import tpu as pltpu
from jax.experimental.pallas import tpu_sc as plsc

_ROWS, _COLS = 1024, 100000
_SC_COLS = 99328
_NC, _NS = 2, 16
_NW = _NC * _NS
_RPW = _ROWS // _NW
_CHUNKS = (25600, 25600, 25600, 22528)
_OFFS = (0, 25600, 51200, 76800)
_UNROLL = 16
_NEG = float("-inf")
_GATHER_DNUMS = lax.GatherDimensionNumbers(
    offset_dims=(), collapsed_slice_dims=(0,), start_index_map=(0,)
)


def _rot(v, iot, sh):
    idx = lax.rem(iot + sh, jnp.full((16,), 16, dtype=jnp.int32))
    return lax.gather(
        v,
        idx[:, None],
        dimension_numbers=_GATHER_DNUMS,
        slice_sizes=(1,),
        mode=lax.GatherScatterMode.PROMISE_IN_BOUNDS,
    )


def _allmax(v, iot):
    for sh in (8, 4, 2, 1):
        v = jnp.maximum(v, _rot(v, iot, sh))
    return v


def _sc_body(x_hbm, y_hbm, outm_hbm, outc_hbm, ybuf, b0, b1, b2, b3, mbuf, cbuf, sems):
    bufs = (b0, b1, b2, b3)
    c = lax.axis_index("c")
    s = lax.axis_index("s")
    wid = s * _NC + c
    base = pl.multiple_of(wid * _RPW, _RPW)
    pltpu.sync_copy(y_hbm.at[pl.ds(base, _RPW)], ybuf)

    def start(j, row):
        pltpu.make_async_copy(
            x_hbm.at[base + row, pl.ds(_OFFS[j], _CHUNKS[j])],
            bufs[j].at[pl.ds(0, _CHUNKS[j])],
            sems.at[j],
        ).start()

    for j in range(4):
        start(j, 0)

    iot = lax.iota(jnp.int32, 16)
    neg = jnp.float32(_NEG)
    negv = jnp.full((16,), _NEG, dtype=jnp.float32)
    sixteen = jnp.full((16,), 16, dtype=jnp.int32)

    def row_body(k, carry):
        om0, om1, oc0, oc1 = carry
        lane = lax.rem(k, 16)
        ybase = pl.multiple_of((k // 16) * 16, 16)
        yv = ybuf[pl.ds(ybase, 16)]
        y_r = _rot(yv, iot, lane)[0]
        acc = negv
        cvec = negv
        for j in range(4):
            size = _CHUNKS[j]
            pltpu.make_async_copy(
                x_hbm.at[base + k, pl.ds(_OFFS[j], size)],
                bufs[j].at[pl.ds(0, size)],
                sems.at[j],
            ).wait()
            off = y_r - _OFFS[j]
            is_mine = (off >= 0) & (off < size)
            offc = jnp.clip(off, 0, size - 1)
            vbase = pl.multiple_of((offc // 16) * 16, 16)
            lane2 = offc - vbase
            v16 = bufs[j][pl.ds(vbase, 16)]
            lane2m = jnp.where(is_mine, lane2, jnp.int32(-100))
            mm = iot == lane2m
            cvec = jnp.where(mm, v16, cvec)
            bufs[j][pl.ds(vbase, 16)] = jnp.where(mm, neg, v16)
            n_it = size // 16 // _UNROLL

            def vec_body(i, a, j=j):
                for u in range(_UNROLL):
                    a = jnp.maximum(a, bufs[j][pl.ds((i * _UNROLL + u) * 16, 16)])
                return a

            acc = lax.fori_loop(0, n_it, vec_body, acc)

            @pl.when(k + 1 < _RPW)
            def _(j=j, k=k):
                start(j, k + 1)

        bestv = _allmax(acc, iot)
        correctv = _allmax(cvec, iot)
        kv = jnp.zeros((16,), jnp.int32) + k
        hit0 = kv == iot
        hit1 = kv == iot + sixteen
        om0 = jnp.where(hit0, bestv, om0)
        om1 = jnp.where(hit1, bestv, om1)
        oc0 = jnp.where(hit0, correctv, oc0)
        oc1 = jnp.where(hit1, correctv, oc1)
        return (om0, om1, oc0, oc1)

    zero16 = jnp.zeros((16,), jnp.float32)
    om0, om1, oc0, oc1 = lax.fori_loop(
        0, _RPW, row_body, (zero16, zero16, zero16, zero16)
    )
    mbuf[pl.ds(0, 16)] = om0
    mbuf[pl.ds(16, 16)] = om1
    cbuf[pl.ds(0, 16)] = oc0
    cbuf[pl.ds(16, 16)] = oc1
    pltpu.sync_copy(mbuf, outm_hbm.at[pl.ds(base, _RPW)])
    pltpu.sync_copy(cbuf, outc_hbm.at[pl.ds(base, _RPW)])


_MESH = plsc.VectorSubcoreMesh(
    core_axis_name="c", subcore_axis_name="s", num_cores=_NC, num_subcores=_NS
)

_TC_BLOCK_ROWS = 64
_TC_GRID = _ROWS // _TC_BLOCK_ROWS
_TC_COL_BLOCK = 1024
_TC_COL_INDEX = _SC_COLS // _TC_COL_BLOCK


def _tc_tail_kernel(y_ref, x_ref, m_ref, c_ref):
    x = x_ref[...]
    yb = y_ref[0, 0, :]
    col = _SC_COLS + jax.lax.broadcasted_iota(jnp.int32, x.shape, 1)
    is_y = col == yb[:, None]
    valid = col < _COLS
    neg = jnp.float32(-jnp.inf)
    m_ref[0, 0, :] = jnp.max(jnp.where(valid & (~is_y), x, neg), axis=-1)
    c_ref[0, 0, :] = jnp.max(jnp.where(valid & is_y, x, neg), axis=-1)


@jax.jit
def kernel(logits, y):
    y32 = y.astype(jnp.int32)
    m_sc, c_sc = pl.kernel(
        _sc_body,
        out_type=(
            jax.ShapeDtypeStruct((_ROWS,), jnp.float32),
            jax.ShapeDtypeStruct((_ROWS,), jnp.float32),
        ),
        mesh=_MESH,
        scratch_types=[
            pltpu.VMEM((_RPW,), jnp.int32),
            pltpu.VMEM((25600,), jnp.float32),
            pltpu.VMEM((25600,), jnp.float32),
            pltpu.VMEM((25600,), jnp.float32),
            pltpu.VMEM((25600,), jnp.float32),
            pltpu.VMEM((_RPW,), jnp.float32),
            pltpu.VMEM((_RPW,), jnp.float32),
            pltpu.SemaphoreType.DMA((4,)),
        ],
    )(logits, y32)

    y3 = y32.reshape(_TC_GRID, 1, _TC_BLOCK_ROWS)
    m_tc, c_tc = pl.pallas_call(
        _tc_tail_kernel,
        grid=(_TC_GRID,),
        in_specs=[
            pl.BlockSpec((1, 1, _TC_BLOCK_ROWS), lambda i: (i, 0, 0)),
            pl.BlockSpec((_TC_BLOCK_ROWS, _TC_COL_BLOCK), lambda i: (i, _TC_COL_INDEX)),
        ],
        out_specs=[
            pl.BlockSpec((1, 1, _TC_BLOCK_ROWS), lambda i: (i, 0, 0)),
            pl.BlockSpec((1, 1, _TC_BLOCK_ROWS), lambda i: (i, 0, 0)),
        ],
        out_shape=[
            jax.ShapeDtypeStruct((_TC_GRID, 1, _TC_BLOCK_ROWS), jnp.float32),
            jax.ShapeDtypeStruct((_TC_GRID, 1, _TC_BLOCK_ROWS), jnp.float32),
        ],
    )(y3, logits)
    m_tc = m_tc.reshape(_ROWS)
    c_tc = c_tc.reshape(_ROWS)

    correct = jnp.maximum(c_sc, c_tc)
    best_other = jnp.maximum(m_sc, m_tc)
    return jnp.maximum(correct - best_other, jnp.float32(0.0))
